# baseline (speedup 1.0000x reference)
import functools

import jax
import jax.numpy as jnp
from jax import lax
from jax.experimental import pallas as pl
from jax.experimental.pallas import tpu as pltpu
from jax.experimental.pallas import tpu_sc as plsc

_BATCH = 16384
_HIST = 50
_DIM = 64
_N = _BATCH * _HIST
_TPAIR = 500000

_NUM_CORES = 2
_NUM_SUBCORES = 16
_NW = _NUM_CORES * _NUM_SUBCORES
_NBT = _BATCH // 128 // _NW
_NB = 2
_L = 16

_mesh = plsc.VectorSubcoreMesh(core_axis_name="c", subcore_axis_name="s")


@functools.partial(
    pl.kernel,
    mesh=_mesh,
    out_type=jax.ShapeDtypeStruct((_HIST, _DIM, _BATCH), jnp.float32),
    scratch_types=[
        pltpu.VMEM((_HIST, 128), jnp.int32),
        pltpu.VMEM((_NB, 128), jnp.int32),
        pltpu.VMEM((_NB, 128), jnp.int32),
        pltpu.VMEM((_NB, 128, 128), jnp.float32),
        pltpu.VMEM((_NB, _DIM, 128), jnp.float32),
        pltpu.SemaphoreType.DMA((_NB,)),
        pltpu.SemaphoreType.DMA((_NB,)),
    ],
    compiler_params=pltpu.CompilerParams(
        use_tc_tiling_on_sc=True, needs_layout_passes=False
    ),
)
def _gather_t(idxt_hbm, table_hbm, out_hbm,
              idx_v, offs_v, colb_v, rows_v, tr_v, sem_g, sem_w):
    wid = lax.axis_index("s") * _NUM_CORES + lax.axis_index("c")

    def bt_body(bt, carry0):
        boff = pl.multiple_of((wid * _NBT + bt) * 128, 128)
        pltpu.sync_copy(idxt_hbm.at[:, pl.ds(boff, 128)], idx_v)

        def prep(h, b):
            for k in range(128 // _L):
                iv = idx_v[h, pl.ds(k * _L, _L)]
                offs_v.at[b][pl.ds(k * _L, _L)] = lax.shift_right_logical(iv, 1)
                colb_v.at[b][pl.ds(k * _L, _L)] = lax.bitwise_and(iv, 1) * _DIM

        def gather_start(b):
            pltpu.async_copy(
                table_hbm.at[offs_v.at[b]], rows_v.at[b], sem_g.at[b]
            )

        def gather_wait(b):
            pltpu.make_async_copy(
                table_hbm.at[offs_v.at[b]], rows_v.at[b], sem_g.at[b]
            ).wait()

        def transpose(b):
            for k in range(128 // _L):
                b_ids = lax.iota(jnp.int32, _L) + k * _L
                cb = colb_v.at[b][pl.ds(k * _L, _L)]
                for d in range(_DIM):
                    x = plsc.load_gather(rows_v.at[b], [b_ids, cb + d])
                    tr_v.at[b][d, pl.ds(k * _L, _L)] = x

        def write_start(h, b):
            pltpu.async_copy(
                tr_v.at[b], out_hbm.at[h, :, pl.ds(boff, 128)], sem_w.at[b]
            )

        def write_wait(b):
            pltpu.make_async_copy(
                tr_v.at[b], out_hbm.at[0, :, pl.ds(boff, 128)], sem_w.at[b]
            ).wait()

        def body(t, carry):
            for b in range(_NB):
                h = t * _NB + b

                @pl.when(t > 0)
                def _():
                    write_wait(b)

                prep(h, b)
                gather_start(b)

                pb = (b - 1) % _NB
                if b > 0:
                    gather_wait(pb)
                    transpose(pb)
                    write_start(h - 1, pb)
                else:
                    @pl.when(t > 0)
                    def _():
                        gather_wait(pb)
                        transpose(pb)
                        write_start(h - 1, pb)
            return carry

        lax.fori_loop(0, _HIST // _NB, body, 0)

        lb = (_HIST - 1) % _NB
        gather_wait(lb)
        transpose(lb)
        write_start(_HIST - 1, lb)
        for b in range(_NB):
            write_wait(b)
        return carry0

    lax.fori_loop(0, _NBT, bt_body, 0)


def kernel(input_sentence, table):
    idxt = input_sentence.T
    table2 = table.reshape(_TPAIR, 2 * _DIM)
    out2 = _gather_t(idxt, table2)
    return out2.transpose(2, 0, 1)


# speedup vs baseline: 1.5451x; 1.5451x over previous
import functools

import jax
import jax.numpy as jnp
from jax import lax
from jax.experimental import pallas as pl
from jax.experimental.pallas import tpu as pltpu
from jax.experimental.pallas import tpu_sc as plsc

_BATCH = 16384
_HIST = 50
_DIM = 64

_NUM_CORES = 2
_NUM_SUBCORES = 16
_NW = _NUM_CORES * _NUM_SUBCORES
_SENT_W = _BATCH // _NW
_CH_S = 8
_CHUNK = _CH_S * _HIST
_NCHUNK = _SENT_W // _CH_S
_NB = 2

_mesh = plsc.VectorSubcoreMesh(core_axis_name="c", subcore_axis_name="s")


@functools.partial(
    pl.kernel,
    mesh=_mesh,
    out_type=jax.ShapeDtypeStruct((_BATCH * _HIST, _DIM), jnp.float32),
    scratch_types=[
        pltpu.VMEM((_NCHUNK * _CHUNK,), jnp.int32),
        pltpu.VMEM((_NB, _CHUNK, _DIM), jnp.float32),
        pltpu.SemaphoreType.DMA((_NB,)),
        pltpu.SemaphoreType.DMA((_NB,)),
    ],
    compiler_params=pltpu.CompilerParams(use_tc_tiling_on_sc=False),
)
def _gather_rows(idx_hbm, table_hbm, out_hbm, idx_v, rows_v, sem_g, sem_w):
    wid = lax.axis_index("s") * _NUM_CORES + lax.axis_index("c")
    base = wid * _SENT_W

    pltpu.sync_copy(idx_hbm.at[pl.ds(base * _HIST, _NCHUNK * _CHUNK)], idx_v)

    def gather_start(g, b):
        pltpu.async_copy(
            table_hbm.at[idx_v.at[pl.ds(g * _CHUNK, _CHUNK)]],
            rows_v.at[b],
            sem_g.at[b],
        )

    def gather_wait(b):
        pltpu.make_async_copy(
            table_hbm.at[idx_v.at[pl.ds(0, _CHUNK)]],
            rows_v.at[b],
            sem_g.at[b],
        ).wait()

    def write_start(g, b):
        pltpu.async_copy(
            rows_v.at[b],
            out_hbm.at[pl.ds((base + g * _CH_S) * _HIST, _CHUNK), :],
            sem_w.at[b],
        )

    def write_wait(b):
        pltpu.make_async_copy(
            rows_v.at[b],
            out_hbm.at[pl.ds(base * _HIST, _CHUNK), :],
            sem_w.at[b],
        ).wait()

    def body(t, carry):
        for b in range(_NB):
            g = t * _NB + b
            @pl.when(t > 0)
            def _():
                write_wait(b)

            gather_start(g, b)

            pb = (b - 1) % _NB
            if b > 0:
                gather_wait(pb)
                write_start(g - 1, pb)
            else:
                @pl.when(t > 0)
                def _():
                    gather_wait(pb)
                    write_start(g - 1, pb)
        return carry

    lax.fori_loop(0, _NCHUNK // _NB, body, 0)

    last = _NCHUNK - 1
    lb = last % _NB
    gather_wait(lb)
    write_start(last, lb)
    for b in range(_NB):
        write_wait(b)


def kernel(input_sentence, table):
    idx = input_sentence.reshape(_BATCH * _HIST).astype(jnp.int32)
    out = _gather_rows(idx, table)
    return out.reshape(_BATCH, _HIST, _DIM)
